# baseline (speedup 1.0000x reference)
import functools

import jax
import jax.numpy as jnp
from jax import lax
from jax.experimental import pallas as pl
from jax.experimental.pallas import tpu as pltpu
from jax.experimental.pallas import tpu_sc as plsc

CHUNK = 128


@functools.lru_cache(maxsize=None)
def _build(BATCH, SEQ, V, D):
    info = plsc.get_sparse_core_info()
    nw = info.num_cores * info.num_subcores
    b_per_w = (BATCH * SEQ) // nw
    n_chunks = b_per_w // CHUNK
    w_per_row = SEQ // b_per_w
    mesh = plsc.VectorSubcoreMesh(core_axis_name="c", subcore_axis_name="s")

    @functools.partial(
        pl.kernel,
        mesh=mesh,
        out_type=jax.ShapeDtypeStruct((BATCH, SEQ, D), jnp.float32),
        scratch_types=[
            pltpu.VMEM((b_per_w,), jnp.int32),
            pltpu.VMEM((n_chunks, CHUNK, D), jnp.float32),
            pltpu.SemaphoreType.DMA((n_chunks,)),
            pltpu.SemaphoreType.DMA((n_chunks,)),
            pltpu.SemaphoreType.DMA,
        ],
    )
    def gather_kernel(
        idx_hbm, table_hbm, out_hbm, idx_v, rows_v, isem, gsem, osem
    ):
        wid = lax.axis_index("s") * info.num_cores + lax.axis_index("c")
        row = wid // w_per_row
        col = (wid % w_per_row) * b_per_w
        idx_loads = [
            pltpu.async_copy(
                idx_hbm.at[row, pl.ds(col + j * CHUNK, CHUNK)],
                idx_v.at[pl.ds(j * CHUNK, CHUNK)],
                isem.at[j],
            )
            for j in range(n_chunks)
        ]
        gathers = []
        for j in range(n_chunks):
            idx_loads[j].wait()
            gathers.append(
                pltpu.async_copy(
                    table_hbm.at[idx_v.at[pl.ds(j * CHUNK, CHUNK)]],
                    rows_v.at[j],
                    gsem.at[j],
                )
            )
        writes = []
        for j in range(n_chunks):
            gathers[j].wait()
            writes.append(
                pltpu.async_copy(
                    rows_v.at[j],
                    out_hbm.at[row, pl.ds(col + j * CHUNK, CHUNK)],
                    osem,
                )
            )
        for w in writes:
            w.wait()

    return gather_kernel


def kernel(x, table):
    BATCH, SEQ = x.shape
    V, D = table.shape
    return _build(BATCH, SEQ, V, D)(x.astype(jnp.int32), table)


# speedup vs baseline: 1.0085x; 1.0085x over previous
import functools

import jax
import jax.numpy as jnp
from jax import lax
from jax.experimental import pallas as pl
from jax.experimental.pallas import tpu as pltpu
from jax.experimental.pallas import tpu_sc as plsc

CHUNK = 512


@functools.lru_cache(maxsize=None)
def _build(BATCH, SEQ, V, D):
    info = plsc.get_sparse_core_info()
    nw = info.num_cores * info.num_subcores
    b_per_w = (BATCH * SEQ) // nw
    n_chunks = b_per_w // CHUNK
    w_per_row = SEQ // b_per_w
    mesh = plsc.VectorSubcoreMesh(core_axis_name="c", subcore_axis_name="s")

    @functools.partial(
        pl.kernel,
        mesh=mesh,
        out_type=jax.ShapeDtypeStruct((BATCH, SEQ, D), jnp.float32),
        scratch_types=[
            pltpu.VMEM((b_per_w,), jnp.int32),
            pltpu.VMEM((n_chunks, CHUNK, D), jnp.float32),
            pltpu.SemaphoreType.DMA((n_chunks,)),
            pltpu.SemaphoreType.DMA((n_chunks,)),
            pltpu.SemaphoreType.DMA,
        ],
    )
    def gather_kernel(
        idx_hbm, table_hbm, out_hbm, idx_v, rows_v, isem, gsem, osem
    ):
        wid = lax.axis_index("s") * info.num_cores + lax.axis_index("c")
        row = wid // w_per_row
        col = (wid % w_per_row) * b_per_w
        idx_loads = [
            pltpu.async_copy(
                idx_hbm.at[row, pl.ds(col + j * CHUNK, CHUNK)],
                idx_v.at[pl.ds(j * CHUNK, CHUNK)],
                isem.at[j],
            )
            for j in range(n_chunks)
        ]
        gathers = []
        for j in range(n_chunks):
            idx_loads[j].wait()
            gathers.append(
                pltpu.async_copy(
                    table_hbm.at[idx_v.at[pl.ds(j * CHUNK, CHUNK)]],
                    rows_v.at[j],
                    gsem.at[j],
                )
            )
        writes = []
        for j in range(n_chunks):
            gathers[j].wait()
            writes.append(
                pltpu.async_copy(
                    rows_v.at[j],
                    out_hbm.at[row, pl.ds(col + j * CHUNK, CHUNK)],
                    osem,
                )
            )
        for w in writes:
            w.wait()

    return gather_kernel


def kernel(x, table):
    BATCH, SEQ = x.shape
    V, D = table.shape
    return _build(BATCH, SEQ, V, D)(x.astype(jnp.int32), table)
